# Initial kernel scaffold; baseline (speedup 1.0000x reference)
import jax
import jax.numpy as jnp
from jax.experimental import pallas as pl


def kernel(observation, first_transition, executing_option, Wm, Wmv, Wt, Wp, Wv):
    raise NotImplementedError("write your pallas kernel here")


# speedup vs baseline: 1.7944x; 1.7944x over previous
import jax
import jax.numpy as jnp
from jax.experimental import pallas as pl

_N = 4096
_D = 1024
_E = 8
_A = 16
_TILE = 512


def _fused_kernel(obs_ref, wp_ref, ws_ref, ft_ref, eo_ref,
                  actions_ref, values_ref, logp_ref, newopt_ref,
                  mv_ref, mlp_ref, tp_ref):
    obs = obs_ref[...]
    act_all = jax.lax.dot_general(
        obs, wp_ref[...], (((1,), (0,)), ((), ())),
        preferred_element_type=jnp.float32)
    small = jax.lax.dot_general(
        obs, ws_ref[...], (((1,), (0,)), ((), ())),
        preferred_element_type=jnp.float32)

    meta_logits = small[:, 0:_E]
    term_logits = small[:, _E:2 * _E]
    vals_all = small[:, 2 * _E:3 * _E]
    meta_values = small[:, 3 * _E]

    ft = ft_ref[...]
    eo = eo_ref[...]

    t = obs.shape[0]
    lane8 = jax.lax.broadcasted_iota(jnp.int32, (t, _E), 1)

    meta_actions = jnp.argmax(meta_logits, axis=-1).astype(jnp.int32)
    mmax = jnp.max(meta_logits, axis=-1)
    mlse = mmax + jnp.log(jnp.sum(jnp.exp(meta_logits - mmax[:, None]), axis=-1))
    meta_log_probs = mmax - mlse

    term_sel = jnp.sum(jnp.where(lane8 == eo[:, None], term_logits, 0.0), axis=-1)
    term_prob = jax.nn.sigmoid(term_sel)
    requires_new = jnp.logical_or(term_prob > 0.5, ft != 0)
    new_opt = jnp.where(requires_new, meta_actions, eo)
    term_prob = jnp.where(ft != 0, 0.0, term_prob)

    values = jnp.sum(jnp.where(lane8 == new_opt[:, None], vals_all, 0.0), axis=-1)

    lane_e = jax.lax.broadcasted_iota(jnp.int32, (t, _E * _A), 1) // _A
    sel_mask = lane_e == new_opt[:, None]
    neg = jnp.float32(jnp.finfo(jnp.float32).min)
    sel = jnp.where(sel_mask, act_all, neg)
    g = jnp.argmax(sel, axis=-1).astype(jnp.int32)
    actions = g - new_opt * _A
    amax = jnp.max(sel, axis=-1)
    alse = amax + jnp.log(
        jnp.sum(jnp.where(sel_mask, jnp.exp(act_all - amax[:, None]), 0.0), axis=-1))
    log_probs = amax - alse

    actions_ref[...] = actions
    values_ref[...] = values
    logp_ref[...] = log_probs
    newopt_ref[...] = new_opt
    mv_ref[...] = meta_values
    mlp_ref[...] = meta_log_probs
    tp_ref[...] = term_prob


def kernel(observation, first_transition, executing_option, Wm, Wmv, Wt, Wp, Wv):
    n, d = observation.shape
    e = Wm.shape[1]
    a = Wp.shape[2]
    wp2d = Wp.transpose(1, 0, 2).reshape(d, e * a)
    wsmall = jnp.concatenate(
        [Wm, Wt, Wv[..., 0].T, Wmv, jnp.zeros((d, 32 - 3 * e - 1), jnp.float32)],
        axis=1)
    ft = first_transition.astype(jnp.int32)
    eo = executing_option.astype(jnp.int32)

    grid = (n // _TILE,)
    row_spec = pl.BlockSpec((_TILE,), lambda i: (i,))
    out_specs = [row_spec] * 7
    out_shapes = [
        jax.ShapeDtypeStruct((n,), jnp.int32),
        jax.ShapeDtypeStruct((n,), jnp.float32),
        jax.ShapeDtypeStruct((n,), jnp.float32),
        jax.ShapeDtypeStruct((n,), jnp.int32),
        jax.ShapeDtypeStruct((n,), jnp.float32),
        jax.ShapeDtypeStruct((n,), jnp.float32),
        jax.ShapeDtypeStruct((n,), jnp.float32),
    ]
    outs = pl.pallas_call(
        _fused_kernel,
        grid=grid,
        in_specs=[
            pl.BlockSpec((_TILE, d), lambda i: (i, 0)),
            pl.BlockSpec((d, e * a), lambda i: (0, 0)),
            pl.BlockSpec((d, 32), lambda i: (0, 0)),
            row_spec,
            row_spec,
        ],
        out_specs=out_specs,
        out_shape=out_shapes,
    )(observation, wp2d, wsmall, ft, eo)
    return tuple(outs)
